# baseline (speedup 1.0000x reference)
import functools
import numpy as np
import jax
import jax.numpy as jnp
from jax.experimental import pallas as pl
from jax.experimental.pallas import tpu as pltpu

_IMG_H = 640.0
_IMG_W = 640.0
_STRIDES = (8, 16, 32)
_SCORE_THR = 0.001
_IOU_THR = 0.65
_MAX_DET = 200
_NUM_CANDS = 2048
_NUM_CLASSES = 80


def _anchors_np():
    grids = []
    strl = []
    for stride in _STRIDES:
        h = int(_IMG_H) // stride
        w = int(_IMG_W) // stride
        yv, xv = np.meshgrid(np.arange(h), np.arange(w), indexing='ij')
        grid = np.stack((xv, yv), 2).reshape(-1, 2)
        grids.append(grid)
        strl.append(np.full((grid.shape[0], 1), stride))
    s = np.concatenate(strl, 0).astype(np.float32)
    off = s * np.concatenate(grids, 0).astype(np.float32)
    xc = off[:, 0:1]
    yc = off[:, 1:2]
    return np.concatenate(
        [(2 * yc - s) / 2, (2 * xc - s) / 2, (2 * yc + s) / 2, (2 * xc + s) / 2], -1)


def _nms_kernel(ty, tx, th, tw, ay1, ax1, ay2, ax2, lab_f, sc,
                oy1, ox1, oy2, ox2, osc, olab):
    B, K = sc.shape

    a_h = ay2[...] - ay1[...]
    a_w = ax2[...] - ax1[...]
    a_yc = ay1[...] + 0.5 * a_h
    a_xc = ax1[...] + 0.5 * a_w
    yc = ty[...] * a_h + a_yc
    xc = tx[...] * a_w + a_xc
    hh = jnp.exp(th[...]) * a_h
    ww = jnp.exp(tw[...]) * a_w
    y1 = jnp.clip(yc - 0.5 * hh, 0.0, _IMG_H)
    x1 = jnp.clip(xc - 0.5 * ww, 0.0, _IMG_W)
    y2 = jnp.clip(yc + 0.5 * hh, 0.0, _IMG_H)
    x2 = jnp.clip(xc + 0.5 * ww, 0.0, _IMG_W)

    lab = lab_f[...]
    off = lab * 1e4
    Y1 = y1 + off
    X1 = x1 + off
    Y2 = y2 + off
    X2 = x2 + off
    areas = (Y2 - Y1) * (X2 - X1)

    iota_k = jax.lax.broadcasted_iota(jnp.int32, (B, K), 1)
    iota_o = jax.lax.broadcasted_iota(jnp.int32, (1, 256), 1)

    def body(i, state):
        rem, ay1a, ax1a, ay2a, ax2a, asca, alaba = state
        v = jnp.max(rem, axis=1, keepdims=True)
        pos = jnp.where(rem == v, iota_k, K)
        jmin = jnp.min(pos, axis=1, keepdims=True)
        onehot = (iota_k == jmin).astype(jnp.float32)

        s_y1 = jnp.sum(y1 * onehot, axis=1, keepdims=True)
        s_x1 = jnp.sum(x1 * onehot, axis=1, keepdims=True)
        s_y2 = jnp.sum(y2 * onehot, axis=1, keepdims=True)
        s_x2 = jnp.sum(x2 * onehot, axis=1, keepdims=True)
        s_lab = jnp.sum(lab * onehot, axis=1, keepdims=True)

        s_off = s_lab * 1e4
        SY1 = s_y1 + s_off
        SX1 = s_x1 + s_off
        SY2 = s_y2 + s_off
        SX2 = s_x2 + s_off
        s_area = (SY2 - SY1) * (SX2 - SX1)

        iy1 = jnp.maximum(SY1, Y1)
        ix1 = jnp.maximum(SX1, X1)
        iy2 = jnp.minimum(SY2, Y2)
        ix2 = jnp.minimum(SX2, X2)
        inter = jnp.clip(iy2 - iy1, 0.0) * jnp.clip(ix2 - ix1, 0.0)
        union = s_area + areas - inter
        iou = inter / jnp.maximum(union, 1e-9)
        rem = jnp.where(jnp.logical_or(iou > _IOU_THR, iota_k == jmin), -1.0, rem)

        valid = (v > 0.0).astype(jnp.float32)
        lane = (iota_o == i).astype(jnp.float32)
        ay1a = ay1a + (valid * s_y1) * lane
        ax1a = ax1a + (valid * s_x1) * lane
        ay2a = ay2a + (valid * s_y2) * lane
        ax2a = ax2a + (valid * s_x2) * lane
        asca = asca + (valid * v) * lane
        alaba = alaba + (valid * s_lab) * lane
        return rem, ay1a, ax1a, ay2a, ax2a, asca, alaba

    z = jnp.zeros((B, 256), jnp.float32)
    state = (sc[...], z, z, z, z, z, z)
    state = jax.lax.fori_loop(0, _MAX_DET, body, state)
    _, ay1a, ax1a, ay2a, ax2a, asca, alaba = state
    oy1[...] = ay1a
    ox1[...] = ax1a
    oy2[...] = ay2a
    ox2[...] = ax2a
    osc[...] = asca
    olab[...] = alaba


_ROWS = 5256
_NFLAT = 8400 * _NUM_CLASSES
_THR_BITS = int(np.float32(_SCORE_THR).view(np.int32))
_ONE_BITS = int(np.float32(1.0).view(np.int32))
_BIG = np.int32(2 ** 30)


def _select_kernel(x_ref, sc_ref, idx_ref, midx_ref):
    r_iota = jax.lax.broadcasted_iota(jnp.int32, (_ROWS, 128), 0)
    c_iota = jax.lax.broadcasted_iota(jnp.int32, (_ROWS, 128), 1)
    gidx = r_iota * 128 + c_iota

    def bis_body(_, st):
        lo, hi, c_hi = st
        mid = (lo + hi) // 2
        t = jax.lax.bitcast_convert_type(mid, jnp.float32)
        c = jnp.sum((x_ref[0] >= t).astype(jnp.int32))
        ge = c >= _NUM_CANDS
        lo = jnp.where(ge, mid, lo)
        hi = jnp.where(ge, hi, mid)
        c_hi = jnp.where(ge, c_hi, c)
        return lo, hi, c_hi

    lo0 = jnp.int32(_THR_BITS)
    hi0 = jnp.int32(_ONE_BITS)
    lo, hi, n_gt = jax.lax.fori_loop(0, 27, bis_body, (lo0, hi0, jnp.int32(0)))
    v = jax.lax.bitcast_convert_type(lo, jnp.float32)
    n_take = _NUM_CANDS - n_gt

    tie_idx = jnp.where(x_ref[0] == v, gidx, _BIG)
    tiemin = jnp.min(tie_idx)

    def cut_bisect():
        def body(_, st):
            clo, chi = st
            m = (clo + chi) // 2
            g = jnp.sum((tie_idx <= m).astype(jnp.int32))
            ge = g >= n_take
            chi = jnp.where(ge, m, chi)
            clo = jnp.where(ge, clo, m)
            return clo, chi
        clo, chi = jax.lax.fori_loop(
            0, 20, body, (jnp.int32(-1), jnp.int32(_NFLAT - 1)))
        return chi

    idx_cut = jax.lax.cond(n_take > 1, cut_bisect, lambda: tiemin)

    x0 = x_ref[0]
    member = jnp.logical_or(x0 > v, jnp.logical_and(x0 == v, gidx <= idx_cut))
    midx_ref[...] = jnp.where(member, gidx, _BIG)

    sc_ref[0, :, :] = jnp.full((_NUM_CANDS, 128), -1.0, jnp.float32)
    idx_ref[0, :, :] = jnp.zeros((_NUM_CANDS, 128), jnp.int32)

    def chunk_body(k, n):
        mi = midx_ref[pl.ds(k * 8, 8), :]
        xc = x_ref[0, pl.ds(k * 8, 8), :]

        def cond(st):
            n2, mi2 = st
            return jnp.min(mi2) < _BIG

        def extract(st):
            n2, mi2 = st
            m = jnp.min(mi2)
            one = mi2 == m
            val = jnp.sum(jnp.where(one, xc, 0.0))
            sc_ref[0, pl.ds(n2, 1), :] = jnp.broadcast_to(val, (1, 128))
            idx_ref[0, pl.ds(n2, 1), :] = jnp.broadcast_to(m, (1, 128))
            mi2 = jnp.where(one, _BIG, mi2)
            return n2 + 1, mi2

        n, _ = jax.lax.while_loop(cond, extract, (n, mi))
        return n

    jax.lax.fori_loop(0, _ROWS // 8, chunk_body, jnp.int32(0))


def kernel(boxes, scores):
    B = boxes.shape[0]
    anchors = jnp.asarray(_anchors_np())

    flat = scores.reshape(B, -1)
    flat_p = jnp.pad(flat, ((0, 0), (0, _ROWS * 128 - _NFLAT)),
                     constant_values=-1.0).reshape(B, _ROWS, 128)
    top_scores, cand_idx = pl.pallas_call(
        _select_kernel,
        grid=(B,),
        in_specs=[pl.BlockSpec((1, _ROWS, 128), lambda b: (b, 0, 0))],
        out_specs=[pl.BlockSpec((1, _NUM_CANDS, 128), lambda b: (b, 0, 0))] * 2,
        out_shape=[
            jax.ShapeDtypeStruct((B, _NUM_CANDS, 128), jnp.float32),
            jax.ShapeDtypeStruct((B, _NUM_CANDS, 128), jnp.int32),
        ],
        scratch_shapes=[pltpu.VMEM((_ROWS, 128), jnp.int32)],
    )(flat_p)
    top_scores = top_scores[:, :, 0]
    cand_idx = cand_idx[:, :, 0]
    box_idx = cand_idx // _NUM_CLASSES
    labels = cand_idx % _NUM_CLASSES

    rel = jnp.take_along_axis(boxes, box_idx[..., None], axis=1)
    anc = jnp.take_along_axis(anchors[None], box_idx[..., None], axis=1)

    args = (
        rel[..., 0], rel[..., 1], rel[..., 2], rel[..., 3],
        anc[..., 0], anc[..., 1], anc[..., 2], anc[..., 3],
        labels.astype(jnp.float32), top_scores,
    )
    outs = pl.pallas_call(
        _nms_kernel,
        out_shape=[jax.ShapeDtypeStruct((B, 256), jnp.float32)] * 6,
    )(*args)
    oy1, ox1, oy2, ox2, osc, olab = outs
    out_boxes = jnp.stack(
        [oy1[:, :_MAX_DET], ox1[:, :_MAX_DET], oy2[:, :_MAX_DET], ox2[:, :_MAX_DET]],
        axis=-1)
    out_scores = osc[:, :_MAX_DET]
    out_labels = olab[:, :_MAX_DET].astype(jnp.int32)
    return out_boxes, out_scores, out_labels


# speedup vs baseline: 1.3511x; 1.3511x over previous
import functools
import numpy as np
import jax
import jax.numpy as jnp
from jax.experimental import pallas as pl
from jax.experimental.pallas import tpu as pltpu

_IMG_H = 640.0
_IMG_W = 640.0
_STRIDES = (8, 16, 32)
_SCORE_THR = 0.001
_IOU_THR = 0.65
_MAX_DET = 200
_NUM_CANDS = 2048
_NUM_CLASSES = 80


def _anchors_np():
    grids = []
    strl = []
    for stride in _STRIDES:
        h = int(_IMG_H) // stride
        w = int(_IMG_W) // stride
        yv, xv = np.meshgrid(np.arange(h), np.arange(w), indexing='ij')
        grid = np.stack((xv, yv), 2).reshape(-1, 2)
        grids.append(grid)
        strl.append(np.full((grid.shape[0], 1), stride))
    s = np.concatenate(strl, 0).astype(np.float32)
    off = s * np.concatenate(grids, 0).astype(np.float32)
    xc = off[:, 0:1]
    yc = off[:, 1:2]
    return np.concatenate(
        [(2 * yc - s) / 2, (2 * xc - s) / 2, (2 * yc + s) / 2, (2 * xc + s) / 2], -1)


def _nms_kernel(ty, tx, th, tw, ay1, ax1, ay2, ax2, lab_f, sc,
                oy1, ox1, oy2, ox2, osc, olab):
    B, K = sc.shape

    a_h = ay2[...] - ay1[...]
    a_w = ax2[...] - ax1[...]
    a_yc = ay1[...] + 0.5 * a_h
    a_xc = ax1[...] + 0.5 * a_w
    yc = ty[...] * a_h + a_yc
    xc = tx[...] * a_w + a_xc
    hh = jnp.exp(th[...]) * a_h
    ww = jnp.exp(tw[...]) * a_w
    y1 = jnp.clip(yc - 0.5 * hh, 0.0, _IMG_H)
    x1 = jnp.clip(xc - 0.5 * ww, 0.0, _IMG_W)
    y2 = jnp.clip(yc + 0.5 * hh, 0.0, _IMG_H)
    x2 = jnp.clip(xc + 0.5 * ww, 0.0, _IMG_W)

    lab = lab_f[...]
    off = lab * 1e4
    Y1 = y1 + off
    X1 = x1 + off
    Y2 = y2 + off
    X2 = x2 + off
    areas = (Y2 - Y1) * (X2 - X1)

    iota_k = jax.lax.broadcasted_iota(jnp.int32, (B, K), 1)
    iota_o = jax.lax.broadcasted_iota(jnp.int32, (1, 256), 1)

    def body(i, state):
        rem, ay1a, ax1a, ay2a, ax2a, asca, alaba = state
        v = jnp.max(rem, axis=1, keepdims=True)
        pos = jnp.where(rem == v, iota_k, K)
        jmin = jnp.min(pos, axis=1, keepdims=True)
        onehot = (iota_k == jmin).astype(jnp.float32)

        s_y1 = jnp.sum(y1 * onehot, axis=1, keepdims=True)
        s_x1 = jnp.sum(x1 * onehot, axis=1, keepdims=True)
        s_y2 = jnp.sum(y2 * onehot, axis=1, keepdims=True)
        s_x2 = jnp.sum(x2 * onehot, axis=1, keepdims=True)
        s_lab = jnp.sum(lab * onehot, axis=1, keepdims=True)

        s_off = s_lab * 1e4
        SY1 = s_y1 + s_off
        SX1 = s_x1 + s_off
        SY2 = s_y2 + s_off
        SX2 = s_x2 + s_off
        s_area = (SY2 - SY1) * (SX2 - SX1)

        iy1 = jnp.maximum(SY1, Y1)
        ix1 = jnp.maximum(SX1, X1)
        iy2 = jnp.minimum(SY2, Y2)
        ix2 = jnp.minimum(SX2, X2)
        inter = jnp.clip(iy2 - iy1, 0.0) * jnp.clip(ix2 - ix1, 0.0)
        union = s_area + areas - inter
        iou = inter / jnp.maximum(union, 1e-9)
        rem = jnp.where(jnp.logical_or(iou > _IOU_THR, iota_k == jmin), -1.0, rem)

        valid = (v > 0.0).astype(jnp.float32)
        lane = (iota_o == i).astype(jnp.float32)
        ay1a = ay1a + (valid * s_y1) * lane
        ax1a = ax1a + (valid * s_x1) * lane
        ay2a = ay2a + (valid * s_y2) * lane
        ax2a = ax2a + (valid * s_x2) * lane
        asca = asca + (valid * v) * lane
        alaba = alaba + (valid * s_lab) * lane
        return rem, ay1a, ax1a, ay2a, ax2a, asca, alaba

    z = jnp.zeros((B, 256), jnp.float32)
    state = (sc[...], z, z, z, z, z, z)
    state = jax.lax.fori_loop(0, _MAX_DET, body, state)
    _, ay1a, ax1a, ay2a, ax2a, asca, alaba = state
    oy1[...] = ay1a
    ox1[...] = ax1a
    oy2[...] = ay2a
    ox2[...] = ax2a
    osc[...] = asca
    olab[...] = alaba


_ROWS = 5280
_NFLAT = 8400 * _NUM_CLASSES
_THR_BITS = int(np.float32(_SCORE_THR).view(np.int32))
_ONE_BITS = int(np.float32(1.0).view(np.int32))
_BIG = np.int32(2 ** 30)


def _select_kernel(x_ref, sc_ref, idx_ref, midx_ref):
    r_iota = jax.lax.broadcasted_iota(jnp.int32, (_ROWS, 128), 0)
    c_iota = jax.lax.broadcasted_iota(jnp.int32, (_ROWS, 128), 1)
    gidx = r_iota * 128 + c_iota

    def bis_body(_, st):
        lo, hi, c_hi = st
        mid = (lo + hi) // 2
        t = jax.lax.bitcast_convert_type(mid, jnp.float32)
        c = jnp.sum((x_ref[0] >= t).astype(jnp.int32))
        ge = c >= _NUM_CANDS
        lo = jnp.where(ge, mid, lo)
        hi = jnp.where(ge, hi, mid)
        c_hi = jnp.where(ge, c_hi, c)
        return lo, hi, c_hi

    lo0 = jnp.int32(_THR_BITS)
    hi0 = jnp.int32(_ONE_BITS)
    lo, hi, n_gt = jax.lax.fori_loop(0, 27, bis_body, (lo0, hi0, jnp.int32(0)))
    v = jax.lax.bitcast_convert_type(lo, jnp.float32)
    n_take = _NUM_CANDS - n_gt

    tie_idx = jnp.where(x_ref[0] == v, gidx, _BIG)
    tiemin = jnp.min(tie_idx)

    def cut_bisect():
        def body(_, st):
            clo, chi = st
            m = (clo + chi) // 2
            g = jnp.sum((tie_idx <= m).astype(jnp.int32))
            ge = g >= n_take
            chi = jnp.where(ge, m, chi)
            clo = jnp.where(ge, clo, m)
            return clo, chi
        clo, chi = jax.lax.fori_loop(
            0, 20, body, (jnp.int32(-1), jnp.int32(_NFLAT - 1)))
        return chi

    idx_cut = jax.lax.cond(n_take > 1, cut_bisect, lambda: tiemin)

    x0 = x_ref[0]
    member = jnp.logical_or(x0 > v, jnp.logical_and(x0 == v, gidx <= idx_cut))
    midx_ref[...] = jnp.where(member, gidx, _BIG)

    sc_ref[0, :, :] = jnp.full((_NUM_CANDS, 128), -1.0, jnp.float32)
    idx_ref[0, :, :] = jnp.zeros((_NUM_CANDS, 128), jnp.int32)

    def chunk_body(k, n):
        mi = midx_ref[pl.ds(k * 32, 32), :]
        xc = x_ref[0, pl.ds(k * 32, 32), :]
        cnt = jnp.sum((mi < _BIG).astype(jnp.int32))

        def extract(_, st):
            n2, mi2 = st
            m = jnp.min(mi2, keepdims=True)[:, :1]
            one = mi2 == m
            val = jnp.sum(jnp.where(one, xc, 0.0), keepdims=True)[:, :1]
            sc_ref[0, pl.ds(n2, 1), :] = jnp.broadcast_to(val, (1, 128))
            idx_ref[0, pl.ds(n2, 1), :] = jnp.broadcast_to(m, (1, 128))
            mi2 = jnp.where(one, _BIG, mi2)
            return n2 + 1, mi2

        n, _ = jax.lax.fori_loop(0, cnt, extract, (n, mi))
        return n

    jax.lax.fori_loop(0, _ROWS // 32, chunk_body, jnp.int32(0))


def kernel(boxes, scores):
    B = boxes.shape[0]
    anchors = jnp.asarray(_anchors_np())

    flat = scores.reshape(B, -1)
    flat_p = jnp.pad(flat, ((0, 0), (0, _ROWS * 128 - _NFLAT)),
                     constant_values=-1.0).reshape(B, _ROWS, 128)
    top_scores, cand_idx = pl.pallas_call(
        _select_kernel,
        grid=(B,),
        in_specs=[pl.BlockSpec((1, _ROWS, 128), lambda b: (b, 0, 0))],
        out_specs=[pl.BlockSpec((1, _NUM_CANDS, 128), lambda b: (b, 0, 0))] * 2,
        out_shape=[
            jax.ShapeDtypeStruct((B, _NUM_CANDS, 128), jnp.float32),
            jax.ShapeDtypeStruct((B, _NUM_CANDS, 128), jnp.int32),
        ],
        scratch_shapes=[pltpu.VMEM((_ROWS, 128), jnp.int32)],
    )(flat_p)
    top_scores = top_scores[:, :, 0]
    cand_idx = cand_idx[:, :, 0]
    box_idx = cand_idx // _NUM_CLASSES
    labels = cand_idx % _NUM_CLASSES

    rel = jnp.take_along_axis(boxes, box_idx[..., None], axis=1)
    anc = jnp.take_along_axis(anchors[None], box_idx[..., None], axis=1)

    args = (
        rel[..., 0], rel[..., 1], rel[..., 2], rel[..., 3],
        anc[..., 0], anc[..., 1], anc[..., 2], anc[..., 3],
        labels.astype(jnp.float32), top_scores,
    )
    outs = pl.pallas_call(
        _nms_kernel,
        out_shape=[jax.ShapeDtypeStruct((B, 256), jnp.float32)] * 6,
    )(*args)
    oy1, ox1, oy2, ox2, osc, olab = outs
    out_boxes = jnp.stack(
        [oy1[:, :_MAX_DET], ox1[:, :_MAX_DET], oy2[:, :_MAX_DET], ox2[:, :_MAX_DET]],
        axis=-1)
    out_scores = osc[:, :_MAX_DET]
    out_labels = olab[:, :_MAX_DET].astype(jnp.int32)
    return out_boxes, out_scores, out_labels


# speedup vs baseline: 1.3610x; 1.0073x over previous
import functools
import numpy as np
import jax
import jax.numpy as jnp
from jax.experimental import pallas as pl
from jax.experimental.pallas import tpu as pltpu

_IMG_H = 640.0
_IMG_W = 640.0
_STRIDES = (8, 16, 32)
_SCORE_THR = 0.001
_IOU_THR = 0.65
_MAX_DET = 200
_NUM_CANDS = 2048
_NUM_CLASSES = 80


def _anchors_np():
    grids = []
    strl = []
    for stride in _STRIDES:
        h = int(_IMG_H) // stride
        w = int(_IMG_W) // stride
        yv, xv = np.meshgrid(np.arange(h), np.arange(w), indexing='ij')
        grid = np.stack((xv, yv), 2).reshape(-1, 2)
        grids.append(grid)
        strl.append(np.full((grid.shape[0], 1), stride))
    s = np.concatenate(strl, 0).astype(np.float32)
    off = s * np.concatenate(grids, 0).astype(np.float32)
    xc = off[:, 0:1]
    yc = off[:, 1:2]
    return np.concatenate(
        [(2 * yc - s) / 2, (2 * xc - s) / 2, (2 * yc + s) / 2, (2 * xc + s) / 2], -1)


def _nms_kernel(ty, tx, th, tw, ay1, ax1, ay2, ax2, lab_f, sc,
                oy1, ox1, oy2, ox2, osc, olab):
    B, K = sc.shape

    a_h = ay2[...] - ay1[...]
    a_w = ax2[...] - ax1[...]
    a_yc = ay1[...] + 0.5 * a_h
    a_xc = ax1[...] + 0.5 * a_w
    yc = ty[...] * a_h + a_yc
    xc = tx[...] * a_w + a_xc
    hh = jnp.exp(th[...]) * a_h
    ww = jnp.exp(tw[...]) * a_w
    y1 = jnp.clip(yc - 0.5 * hh, 0.0, _IMG_H)
    x1 = jnp.clip(xc - 0.5 * ww, 0.0, _IMG_W)
    y2 = jnp.clip(yc + 0.5 * hh, 0.0, _IMG_H)
    x2 = jnp.clip(xc + 0.5 * ww, 0.0, _IMG_W)

    lab = lab_f[...]
    off = lab * 1e4
    Y1 = y1 + off
    X1 = x1 + off
    Y2 = y2 + off
    X2 = x2 + off
    areas = (Y2 - Y1) * (X2 - X1)

    iota_k = jax.lax.broadcasted_iota(jnp.int32, (B, K), 1)
    iota_o = jax.lax.broadcasted_iota(jnp.int32, (1, 256), 1)

    def body(i, state):
        rem, ay1a, ax1a, ay2a, ax2a, asca, alaba = state
        v = jnp.max(rem, axis=1, keepdims=True)
        pos = jnp.where(rem == v, iota_k, K)
        jmin = jnp.min(pos, axis=1, keepdims=True)
        onehot = (iota_k == jmin).astype(jnp.float32)

        s_y1 = jnp.sum(y1 * onehot, axis=1, keepdims=True)
        s_x1 = jnp.sum(x1 * onehot, axis=1, keepdims=True)
        s_y2 = jnp.sum(y2 * onehot, axis=1, keepdims=True)
        s_x2 = jnp.sum(x2 * onehot, axis=1, keepdims=True)
        s_lab = jnp.sum(lab * onehot, axis=1, keepdims=True)

        s_off = s_lab * 1e4
        SY1 = s_y1 + s_off
        SX1 = s_x1 + s_off
        SY2 = s_y2 + s_off
        SX2 = s_x2 + s_off
        s_area = (SY2 - SY1) * (SX2 - SX1)

        iy1 = jnp.maximum(SY1, Y1)
        ix1 = jnp.maximum(SX1, X1)
        iy2 = jnp.minimum(SY2, Y2)
        ix2 = jnp.minimum(SX2, X2)
        inter = jnp.clip(iy2 - iy1, 0.0) * jnp.clip(ix2 - ix1, 0.0)
        union = s_area + areas - inter
        iou = inter / jnp.maximum(union, 1e-9)
        rem = jnp.where(jnp.logical_or(iou > _IOU_THR, iota_k == jmin), -1.0, rem)

        valid = (v > 0.0).astype(jnp.float32)
        lane = (iota_o == i).astype(jnp.float32)
        ay1a = ay1a + (valid * s_y1) * lane
        ax1a = ax1a + (valid * s_x1) * lane
        ay2a = ay2a + (valid * s_y2) * lane
        ax2a = ax2a + (valid * s_x2) * lane
        asca = asca + (valid * v) * lane
        alaba = alaba + (valid * s_lab) * lane
        return rem, ay1a, ax1a, ay2a, ax2a, asca, alaba

    z = jnp.zeros((B, 256), jnp.float32)
    state = (sc[...], z, z, z, z, z, z)
    state = jax.lax.fori_loop(0, _MAX_DET, body, state)
    _, ay1a, ax1a, ay2a, ax2a, asca, alaba = state
    oy1[...] = ay1a
    ox1[...] = ax1a
    oy2[...] = ay2a
    ox2[...] = ax2a
    osc[...] = asca
    olab[...] = alaba


_ROWS = 5280
_NFLAT = 8400 * _NUM_CLASSES
_THR_BITS = int(np.float32(_SCORE_THR).view(np.int32))
_ONE_BITS = int(np.float32(1.0).view(np.int32))
_BIG = np.int32(2 ** 30)


def _select_kernel(x_ref, sc_ref, idx_ref, midx_ref):
    r_iota = jax.lax.broadcasted_iota(jnp.int32, (_ROWS, 128), 0)
    c_iota = jax.lax.broadcasted_iota(jnp.int32, (_ROWS, 128), 1)
    gidx = r_iota * 128 + c_iota

    def bis_body(_, st):
        lo, hi, c_hi = st
        mid = (lo + hi) // 2
        t = jax.lax.bitcast_convert_type(mid, jnp.float32)
        c = jnp.sum((x_ref[0] >= t).astype(jnp.int32), keepdims=True)[:, :1]
        ge = c >= _NUM_CANDS
        lo = jnp.where(ge, mid, lo)
        hi = jnp.where(ge, hi, mid)
        c_hi = jnp.where(ge, c_hi, c)
        return lo, hi, c_hi

    lo0 = jnp.full((1, 1), _THR_BITS, jnp.int32)
    hi0 = jnp.full((1, 1), _ONE_BITS, jnp.int32)
    lo, hi, n_gt = jax.lax.fori_loop(
        0, 27, bis_body, (lo0, hi0, jnp.zeros((1, 1), jnp.int32)))
    v = jax.lax.bitcast_convert_type(lo, jnp.float32)
    n_take = _NUM_CANDS - n_gt

    tie_idx = jnp.where(x_ref[0] == v, gidx, _BIG)
    tiemin = jnp.min(tie_idx, keepdims=True)[:, :1]

    def cut_bisect():
        def body(_, st):
            clo, chi = st
            m = (clo + chi) // 2
            g = jnp.sum((tie_idx <= m).astype(jnp.int32), keepdims=True)[:, :1]
            ge = g >= n_take
            chi = jnp.where(ge, m, chi)
            clo = jnp.where(ge, clo, m)
            return clo, chi
        clo, chi = jax.lax.fori_loop(
            0, 20, body,
            (jnp.full((1, 1), -1, jnp.int32),
             jnp.full((1, 1), _NFLAT - 1, jnp.int32)))
        return chi

    idx_cut = jax.lax.cond(n_take[0, 0] > 1, cut_bisect, lambda: tiemin)

    x0 = x_ref[0]
    member = jnp.logical_or(x0 > v, jnp.logical_and(x0 == v, gidx <= idx_cut))
    midx_ref[...] = jnp.where(member, gidx, _BIG)

    sc_ref[0, :, :] = jnp.full((_NUM_CANDS, 128), -1.0, jnp.float32)
    idx_ref[0, :, :] = jnp.zeros((_NUM_CANDS, 128), jnp.int32)

    def chunk_body(k, n):
        mi = midx_ref[pl.ds(k * 32, 32), :]
        xc = x_ref[0, pl.ds(k * 32, 32), :]
        cnt = jnp.sum((mi < _BIG).astype(jnp.int32))

        def extract(_, st):
            n2, mi2 = st
            m = jnp.min(mi2, keepdims=True)[:, :1]
            one = mi2 == m
            val = jnp.sum(jnp.where(one, xc, 0.0), keepdims=True)[:, :1]
            sc_ref[0, pl.ds(n2, 1), :] = jnp.broadcast_to(val, (1, 128))
            idx_ref[0, pl.ds(n2, 1), :] = jnp.broadcast_to(m, (1, 128))
            mi2 = jnp.where(one, _BIG, mi2)
            return n2 + 1, mi2

        n, _ = jax.lax.fori_loop(0, cnt, extract, (n, mi))
        return n

    jax.lax.fori_loop(0, _ROWS // 32, chunk_body, jnp.int32(0))


def kernel(boxes, scores):
    B = boxes.shape[0]
    anchors = jnp.asarray(_anchors_np())

    flat = scores.reshape(B, -1)
    flat_p = jnp.pad(flat, ((0, 0), (0, _ROWS * 128 - _NFLAT)),
                     constant_values=-1.0).reshape(B, _ROWS, 128)
    top_scores, cand_idx = pl.pallas_call(
        _select_kernel,
        grid=(B,),
        in_specs=[pl.BlockSpec((1, _ROWS, 128), lambda b: (b, 0, 0))],
        out_specs=[pl.BlockSpec((1, _NUM_CANDS, 128), lambda b: (b, 0, 0))] * 2,
        out_shape=[
            jax.ShapeDtypeStruct((B, _NUM_CANDS, 128), jnp.float32),
            jax.ShapeDtypeStruct((B, _NUM_CANDS, 128), jnp.int32),
        ],
        scratch_shapes=[pltpu.VMEM((_ROWS, 128), jnp.int32)],
    )(flat_p)
    top_scores = top_scores[:, :, 0]
    cand_idx = cand_idx[:, :, 0]
    box_idx = cand_idx // _NUM_CLASSES
    labels = cand_idx % _NUM_CLASSES

    rel = jnp.take_along_axis(boxes, box_idx[..., None], axis=1)
    anc = jnp.take_along_axis(anchors[None], box_idx[..., None], axis=1)

    args = (
        rel[..., 0], rel[..., 1], rel[..., 2], rel[..., 3],
        anc[..., 0], anc[..., 1], anc[..., 2], anc[..., 3],
        labels.astype(jnp.float32), top_scores,
    )
    outs = pl.pallas_call(
        _nms_kernel,
        out_shape=[jax.ShapeDtypeStruct((B, 256), jnp.float32)] * 6,
    )(*args)
    oy1, ox1, oy2, ox2, osc, olab = outs
    out_boxes = jnp.stack(
        [oy1[:, :_MAX_DET], ox1[:, :_MAX_DET], oy2[:, :_MAX_DET], ox2[:, :_MAX_DET]],
        axis=-1)
    out_scores = osc[:, :_MAX_DET]
    out_labels = olab[:, :_MAX_DET].astype(jnp.int32)
    return out_boxes, out_scores, out_labels
